# baseline (speedup 1.0000x reference)
import jax
import jax.numpy as jnp
from jax.experimental import pallas as pl

_BN = 2000


def _heads_kernel(x_ref, wc_ref, bc_ref, wb_ref, bb_ref, wi_ref, bi_ref,
                  s_ref, d_ref, i_ref):
    x = x_ref[...]
    s_ref[...] = jnp.dot(x, wc_ref[...],
                         preferred_element_type=jnp.float32) + bc_ref[...]
    d_ref[...] = jnp.dot(x, wb_ref[...],
                         preferred_element_type=jnp.float32) + bb_ref[...]
    i_ref[...] = jnp.dot(x, wi_ref[...],
                         preferred_element_type=jnp.float32) + bi_ref[...]


def kernel(x, W_cls, b_cls, W_bbox, b_bbox, W_iou, b_iou):
    if x.ndim > 2:
        x = x.reshape(x.shape[0], -1)
    n, d = x.shape
    kc = W_cls.shape[0]
    kb = W_bbox.shape[0]
    ki = W_iou.shape[0]

    wc = W_cls.T
    wb = W_bbox.T
    wi = W_iou.T
    bc = b_cls.reshape(1, kc)
    bb = b_bbox.reshape(1, kb)
    bi = b_iou.reshape(1, ki)

    grid = (n // _BN,)
    row_block = lambda i: (i, 0)
    whole = lambda i: (0, 0)

    scores, deltas, iou = pl.pallas_call(
        _heads_kernel,
        grid=grid,
        in_specs=[
            pl.BlockSpec((_BN, d), row_block),
            pl.BlockSpec((d, kc), whole),
            pl.BlockSpec((1, kc), whole),
            pl.BlockSpec((d, kb), whole),
            pl.BlockSpec((1, kb), whole),
            pl.BlockSpec((d, ki), whole),
            pl.BlockSpec((1, ki), whole),
        ],
        out_specs=[
            pl.BlockSpec((_BN, kc), row_block),
            pl.BlockSpec((_BN, kb), row_block),
            pl.BlockSpec((_BN, ki), row_block),
        ],
        out_shape=[
            jax.ShapeDtypeStruct((n, kc), jnp.float32),
            jax.ShapeDtypeStruct((n, kb), jnp.float32),
            jax.ShapeDtypeStruct((n, ki), jnp.float32),
        ],
    )(x, wc, bc, wb, bb, wi, bi)
    return scores, deltas, iou
